# Initial kernel scaffold; baseline (speedup 1.0000x reference)
import jax
import jax.numpy as jnp
from jax.experimental import pallas as pl


def kernel(x, alpha):
    raise NotImplementedError("write your pallas kernel here")


# speedup vs baseline: 3.6960x; 3.6960x over previous
import functools

import jax
import jax.numpy as jnp
from jax import lax
from jax.experimental import pallas as pl
from jax.experimental.pallas import tpu as pltpu
from jax.experimental.pallas import tpu_sc as plsc

B, C, T, K = 64, 256, 2048, 32

L = 16
ROWS_PER_CHUNK = 16
NUM_CHUNKS = B // ROWS_PER_CHUNK


def _selector_dup_kernel(x_hbm, alpha_hbm, z_hbm, w_hbm,
                         arow_v, wrow_v, buf0, buf1, gsem0, gsem1,
                         ssem0, ssem1):
    nc = 2
    wid = lax.axis_index("s") * nc + lax.axis_index("c")

    pltpu.sync_copy(alpha_hbm.at[wid], arow_v)
    iota = lax.iota(jnp.int32, L)
    best_v = arow_v[pl.ds(0, L)]
    best_i = iota
    for j in range(1, C // L):
        v = arow_v[pl.ds(j * L, L)]
        pos = iota + j * L
        upd = v > best_v
        best_v = jnp.where(upd, v, best_v)
        best_i = jnp.where(upd, pos, best_i)
    sk, _ = plsc.sort_key_val(best_v, best_i)
    m = sk[15]
    cand = jnp.where(best_v == m, best_i, jnp.int32(C))
    ck_sorted, _ = plsc.sort_key_val(cand, cand)
    c_k = ck_sorted[0]

    for j in range(C // L):
        pos = iota + j * L
        wrow_v[pl.ds(j * L, L)] = jnp.where(pos == c_k, 1.0, 0.0).astype(
            jnp.float32)
    pltpu.sync_copy(wrow_v, w_hbm.at[wid])

    bufs = (buf0, buf1)
    gsems = (gsem0, gsem1)
    ssems = (ssem0, ssem1)

    def gidx(ch):
        return (iota + ch * ROWS_PER_CHUNK) * C + c_k

    def sidx(ch):
        return (iota + ch * ROWS_PER_CHUNK) * K + wid

    gathers = [None] * NUM_CHUNKS
    scatters = [None] * NUM_CHUNKS
    gathers[0] = pltpu.async_copy(x_hbm.at[gidx(0)], bufs[0], gsems[0])
    for ch in range(NUM_CHUNKS):
        p = ch % 2
        gathers[ch].wait()
        scatters[ch] = pltpu.async_copy(bufs[p], z_hbm.at[sidx(ch)], ssems[p])
        if ch + 1 < NUM_CHUNKS:
            if ch >= 1:
                scatters[ch - 1].wait()
            gathers[ch + 1] = pltpu.async_copy(
                x_hbm.at[gidx(ch + 1)], bufs[(ch + 1) % 2], gsems[(ch + 1) % 2])
    scatters[NUM_CHUNKS - 2].wait()
    scatters[NUM_CHUNKS - 1].wait()


@jax.jit
def _run(x_flat, alpha):
    mesh = plsc.VectorSubcoreMesh(core_axis_name="c", subcore_axis_name="s")
    fn = functools.partial(
        pl.kernel, mesh=mesh,
        compiler_params=pltpu.CompilerParams(needs_layout_passes=False),
        out_type=[
            jax.ShapeDtypeStruct((B * K, T), jnp.float32),
            jax.ShapeDtypeStruct((K, C), jnp.float32),
        ],
        scratch_types=[
            pltpu.VMEM((C,), jnp.float32),
            pltpu.VMEM((C,), jnp.float32),
            pltpu.VMEM((ROWS_PER_CHUNK, T), jnp.float32),
            pltpu.VMEM((ROWS_PER_CHUNK, T), jnp.float32),
            pltpu.SemaphoreType.DMA,
            pltpu.SemaphoreType.DMA,
            pltpu.SemaphoreType.DMA,
            pltpu.SemaphoreType.DMA,
        ],
    )(_selector_dup_kernel)
    return fn(x_flat, alpha)


def kernel(x, alpha):
    z_flat, w_hard = _run(x.reshape(B * C, T), alpha)
    return (z_flat.reshape(B, 1, K, T), w_hard, w_hard)


# speedup vs baseline: 4.0557x; 1.0973x over previous
import functools

import jax
import jax.numpy as jnp
from jax import lax
from jax.experimental import pallas as pl
from jax.experimental.pallas import tpu as pltpu
from jax.experimental.pallas import tpu_sc as plsc

B, C, T, K = 64, 256, 2048, 32

L = 16
NBUF = 3
ROWS_PER_CHUNK = 16
NUM_CHUNKS = B // ROWS_PER_CHUNK


def _selector_dup_kernel(x_hbm, alpha_hbm, z_hbm, w_hbm, w2_hbm,
                         arow_v, wrow_v,
                         buf0, buf1, buf2,
                         gsem0, gsem1, gsem2,
                         ssem0, ssem1, ssem2):
    nc = 2
    wid = lax.axis_index("s") * nc + lax.axis_index("c")

    pltpu.sync_copy(alpha_hbm.at[wid], arow_v)
    iota = lax.iota(jnp.int32, L)
    best_v = arow_v[pl.ds(0, L)]
    best_i = iota
    for j in range(1, C // L):
        v = arow_v[pl.ds(j * L, L)]
        pos = iota + j * L
        upd = v > best_v
        best_v = jnp.where(upd, v, best_v)
        best_i = jnp.where(upd, pos, best_i)
    sk, _ = plsc.sort_key_val(best_v, best_i)
    m = sk[15]
    cand = jnp.where(best_v == m, best_i, jnp.int32(C))
    ck_sorted, _ = plsc.sort_key_val(cand, cand)
    c_k = ck_sorted[0]

    bufs = (buf0, buf1, buf2)
    gsems = (gsem0, gsem1, gsem2)
    ssems = (ssem0, ssem1, ssem2)

    def gidx(ch):
        return (iota + ch * ROWS_PER_CHUNK) * C + c_k

    def sidx(ch):
        return (iota + ch * ROWS_PER_CHUNK) * K + wid

    def gather(ch):
        return pltpu.async_copy(x_hbm.at[gidx(ch)], bufs[ch % NBUF],
                                gsems[ch % NBUF])

    gathers = [None] * NUM_CHUNKS
    scatters = [None] * NUM_CHUNKS
    for ch in range(NBUF - 1):
        gathers[ch] = gather(ch)

    for j in range(C // L):
        pos = iota + j * L
        wrow_v[pl.ds(j * L, L)] = jnp.where(pos == c_k, 1.0, 0.0).astype(
            jnp.float32)
    pltpu.sync_copy(wrow_v, w_hbm.at[wid])
    pltpu.sync_copy(wrow_v, w2_hbm.at[wid])

    for ch in range(NUM_CHUNKS):
        nxt = ch + NBUF - 1
        if nxt < NUM_CHUNKS:
            if ch >= 1:
                scatters[ch - 1].wait()
            gathers[nxt] = gather(nxt)
        gathers[ch].wait()
        scatters[ch] = pltpu.async_copy(
            bufs[ch % NBUF], z_hbm.at[sidx(ch)], ssems[ch % NBUF])
    for ch in range(NUM_CHUNKS - NBUF, NUM_CHUNKS):
        scatters[ch].wait()


@jax.jit
def _run(x_flat, alpha):
    mesh = plsc.VectorSubcoreMesh(core_axis_name="c", subcore_axis_name="s")
    fn = functools.partial(
        pl.kernel, mesh=mesh,
        compiler_params=pltpu.CompilerParams(needs_layout_passes=False),
        out_type=[
            jax.ShapeDtypeStruct((B * K, T), jnp.float32),
            jax.ShapeDtypeStruct((K, C), jnp.float32),
            jax.ShapeDtypeStruct((K, C), jnp.float32),
        ],
        scratch_types=(
            [pltpu.VMEM((C,), jnp.float32)] * 2
            + [pltpu.VMEM((ROWS_PER_CHUNK, T), jnp.float32)] * NBUF
            + [pltpu.SemaphoreType.DMA] * (2 * NBUF)
        ),
    )(_selector_dup_kernel)
    return fn(x_flat, alpha)


def kernel(x, alpha):
    z_flat, w_hard, w_hard2 = _run(x.reshape(B * C, T), alpha)
    return (z_flat.reshape(B, 1, K, T), w_hard, w_hard2)
